# baseline (speedup 1.0000x reference)
import functools

import jax
import jax.numpy as jnp
from jax import lax
from jax.experimental import pallas as pl
from jax.experimental.pallas import tpu as pltpu
from jax.experimental.pallas import tpu_sc as plsc

VOCAB = 100000
EMBED_DIM = 64
NUM_CLASS = 2
BATCH = 4096
BAG_LEN = 50

NP = 16
NC = 2
NS = 16
LANES = 16
NW = NC * NS
BPW = BATCH // NW
CB = 2
CHUNK_IDX = CB * BAG_LEN
NCH = BPW // CB
NBUF = 16

_mesh = plsc.VectorSubcoreMesh(core_axis_name="c", subcore_axis_name="s")

_OROWS = BATCH // 8
_ORPW = _OROWS // NW


@functools.partial(
    pl.kernel,
    mesh=_mesh,
    out_type=jax.ShapeDtypeStruct((_OROWS, 8 * NP), jnp.float32),
    scratch_types=[
        pltpu.VMEM((NCH, CHUNK_IDX), jnp.int32),
        pltpu.VMEM((NBUF, CHUNK_IDX, NP), jnp.float32),
        pltpu.VMEM((_ORPW, 8 * NP), jnp.float32),
        pltpu.VMEM((NP,), jnp.float32),
    ] + [pltpu.SemaphoreType.DMA] * NBUF,
    compiler_params=pltpu.CompilerParams(use_tc_tiling_on_sc=False),
)
def _bag_kernel(idx_hbm, p_hbm, b_hbm, out_hbm,
                idx_v, rows_v, out_v, b_v, *sems):
    wid = lax.axis_index("s") * NC + lax.axis_index("c")
    base = wid * NCH
    pltpu.sync_copy(idx_hbm.at[pl.ds(base, NCH)], idx_v)
    pltpu.sync_copy(b_hbm, b_v)
    bias = b_v[...]

    for bb in range(NBUF):
        pltpu.async_copy(p_hbm.at[idx_v.at[bb]], rows_v.at[bb], sems[bb])

    def body(g, _):
        for bb in range(NBUF):
            ch = g * NBUF + bb
            pltpu.make_async_copy(
                p_hbm.at[idx_v.at[bb]], rows_v.at[bb], sems[bb]
            ).wait()
            for bag in range(CB):
                r0 = bag * BAG_LEN
                acc0 = rows_v[bb, r0, :]
                acc1 = rows_v[bb, r0 + 1, :]
                for l in range(2, BAG_LEN, 2):
                    acc0 = acc0 + rows_v[bb, r0 + l, :]
                    acc1 = acc1 + rows_v[bb, r0 + l + 1, :]
                bg = bb * CB + bag
                out_v[(NBUF * CB // 8) * g + bg // 8, pl.ds((bg % 8) * NP, NP)] = (
                    acc0 + acc1 + bias
                )
            nxt = ch + NBUF

            @pl.when(nxt < NCH)
            def _():
                pltpu.async_copy(
                    p_hbm.at[idx_v.at[nxt]], rows_v.at[bb], sems[bb]
                )

        return 0

    lax.fori_loop(0, NCH // NBUF, body, 0)
    pltpu.sync_copy(out_v, out_hbm.at[pl.ds(wid * _ORPW, _ORPW)])


_BLK = 2048
_GRP = _BLK // 8
_GRID = (VOCAB + _BLK - 1) // _BLK


def _proj_block(tt_ref, w3_ref, o_ref):
    x = tt_ref[...]
    xr = jnp.concatenate(
        [x[:, g * _GRP:(g + 1) * _GRP] for g in range(8)], axis=0
    )
    o_ref[...] = jax.lax.dot_general(
        xr, w3_ref[...], (((0,), (0,)), ((), ())),
        preferred_element_type=jnp.float32,
        precision=jax.lax.Precision.DEFAULT,
    )


_VPAD = _GRID * _BLK


def _tc_project(table_t, w3):
    return pl.pallas_call(
        _proj_block,
        grid=(_GRID,),
        in_specs=[
            pl.BlockSpec((EMBED_DIM, _BLK), lambda i: (0, i)),
            pl.BlockSpec((8 * EMBED_DIM, 8 * NP), lambda i: (0, 0)),
        ],
        out_specs=pl.BlockSpec((_BLK // 8, 8 * NP), lambda i: (i, 0)),
        out_shape=jax.ShapeDtypeStruct((_VPAD // 8, 8 * NP), jnp.float32),
    )(table_t, w3)


def kernel(token_index, table, W, b):
    tok = token_index.astype(jnp.int32)
    tok = (tok // _BLK) * _BLK + (tok % _GRP) * 8 + (tok % _BLK) // _GRP
    idx = jnp.reshape(tok, (BATCH // CB, CHUNK_IDX))
    wt16 = (
        jnp.zeros((EMBED_DIM, NP), jnp.float32)
        .at[:, :NUM_CLASS]
        .set(W.T * jnp.float32(1.0 / BAG_LEN))
    )
    w3 = jnp.kron(jnp.eye(8, dtype=jnp.float32), wt16)
    b_pad = jnp.zeros((NP,), jnp.float32).at[:NUM_CLASS].set(b)
    proj = _tc_project(table.T, w3)
    p_rows = jnp.reshape(proj, (_VPAD, NP))
    out = _bag_kernel(idx, p_rows, b_pad)
    return jnp.reshape(out, (BATCH, NP))[:, :NUM_CLASS]


# speedup vs baseline: 1.4192x; 1.4192x over previous
import functools

import jax
import jax.numpy as jnp
from jax import lax
from jax.experimental import pallas as pl
from jax.experimental.pallas import tpu as pltpu
from jax.experimental.pallas import tpu_sc as plsc

VOCAB = 100000
EMBED_DIM = 64
NUM_CLASS = 2
BATCH = 4096
BAG_LEN = 50

NP = 16
NC = 2
NS = 16
LANES = 16
NW = NC * NS
BPW = BATCH // NW
CB = 2
CHUNK_IDX = CB * BAG_LEN
NCH = BPW // CB
NBUF = 8

_mesh = plsc.VectorSubcoreMesh(core_axis_name="c", subcore_axis_name="s")

_OROWS = BATCH // 8
_ORPW = _OROWS // NW


@functools.partial(
    pl.kernel,
    mesh=_mesh,
    out_type=jax.ShapeDtypeStruct((_OROWS, 8 * NP), jnp.float32),
    scratch_types=[
        pltpu.VMEM((NCH, CHUNK_IDX), jnp.int32),
        pltpu.VMEM((NBUF, CHUNK_IDX, NP), jnp.float32),
        pltpu.VMEM((_ORPW, 8 * NP), jnp.float32),
        pltpu.VMEM((NP,), jnp.float32),
    ] + [pltpu.SemaphoreType.DMA] * NBUF,
    compiler_params=pltpu.CompilerParams(use_tc_tiling_on_sc=False),
)
def _bag_kernel(idx_hbm, p_hbm, b_hbm, out_hbm,
                idx_v, rows_v, out_v, b_v, *sems):
    wid = lax.axis_index("s") * NC + lax.axis_index("c")
    base = wid * NCH
    pltpu.sync_copy(idx_hbm.at[pl.ds(base, NCH)], idx_v)
    pltpu.sync_copy(b_hbm, b_v)
    bias = b_v[...]

    for bb in range(NBUF):
        pltpu.async_copy(p_hbm.at[idx_v.at[bb]], rows_v.at[bb], sems[bb])

    def body(g, _):
        for bb in range(NBUF):
            ch = g * NBUF + bb
            pltpu.make_async_copy(
                p_hbm.at[idx_v.at[bb]], rows_v.at[bb], sems[bb]
            ).wait()
            for bag in range(CB):
                r0 = bag * BAG_LEN
                acc0 = rows_v[bb, r0, :]
                acc1 = rows_v[bb, r0 + 1, :]
                for l in range(2, BAG_LEN, 2):
                    acc0 = acc0 + rows_v[bb, r0 + l, :]
                    acc1 = acc1 + rows_v[bb, r0 + l + 1, :]
                bg = bb * CB + bag
                out_v[(NBUF * CB // 8) * g + bg // 8, pl.ds((bg % 8) * NP, NP)] = (
                    acc0 + acc1 + bias
                )
            nxt = ch + NBUF

            @pl.when(nxt < NCH)
            def _():
                pltpu.async_copy(
                    p_hbm.at[idx_v.at[nxt]], rows_v.at[bb], sems[bb]
                )

        return 0

    lax.fori_loop(0, NCH // NBUF, body, 0)
    pltpu.sync_copy(out_v, out_hbm.at[pl.ds(wid * _ORPW, _ORPW)])


_BLK = 8192
_GRP = _BLK // 8
_GRID = (VOCAB + _BLK - 1) // _BLK


def _proj_block(tt_ref, w3_ref, o_ref):
    x = tt_ref[...]
    xr = jnp.concatenate(
        [x[:, g * _GRP:(g + 1) * _GRP] for g in range(8)], axis=0
    )
    o_ref[...] = jax.lax.dot_general(
        xr, w3_ref[...], (((0,), (0,)), ((), ())),
        preferred_element_type=jnp.float32,
        precision=jax.lax.Precision.DEFAULT,
    )


_VPAD = _GRID * _BLK


def _tc_project(table_t, w3):
    return pl.pallas_call(
        _proj_block,
        grid=(_GRID,),
        in_specs=[
            pl.BlockSpec((EMBED_DIM, _BLK), lambda i: (0, i)),
            pl.BlockSpec((8 * EMBED_DIM, 8 * NP), lambda i: (0, 0)),
        ],
        out_specs=pl.BlockSpec((_BLK // 8, 8 * NP), lambda i: (i, 0)),
        out_shape=jax.ShapeDtypeStruct((_VPAD // 8, 8 * NP), jnp.float32),
    )(table_t, w3)


def kernel(token_index, table, W, b):
    tok = token_index.astype(jnp.int32)
    tok = (tok // _BLK) * _BLK + (tok % _GRP) * 8 + (tok % _BLK) // _GRP
    idx = jnp.reshape(tok, (BATCH // CB, CHUNK_IDX))
    wt16 = (
        jnp.zeros((EMBED_DIM, NP), jnp.float32)
        .at[:, :NUM_CLASS]
        .set(W.T * jnp.float32(1.0 / BAG_LEN))
    )
    w3 = jnp.kron(jnp.eye(8, dtype=jnp.float32), wt16)
    b_pad = jnp.zeros((NP,), jnp.float32).at[:NUM_CLASS].set(b)
    proj = _tc_project(table.T, w3)
    p_rows = jnp.reshape(proj, (_VPAD, NP))
    out = _bag_kernel(idx, p_rows, b_pad)
    return jnp.reshape(out, (BATCH, NP))[:, :NUM_CLASS]


# speedup vs baseline: 1.4948x; 1.0533x over previous
import functools

import jax
import jax.numpy as jnp
from jax import lax
from jax.experimental import pallas as pl
from jax.experimental.pallas import tpu as pltpu
from jax.experimental.pallas import tpu_sc as plsc

VOCAB = 100000
EMBED_DIM = 64
NUM_CLASS = 2
BATCH = 4096
BAG_LEN = 50

NP = 16
NC = 2
NS = 16
LANES = 16
NW = NC * NS
BPW = BATCH // NW
CB = 2
CHUNK_IDX = CB * BAG_LEN
NCH = BPW // CB
NBUF = 8

_mesh = plsc.VectorSubcoreMesh(core_axis_name="c", subcore_axis_name="s")

_OROWS = BATCH // 8
_ORPW = _OROWS // NW


@functools.partial(
    pl.kernel,
    mesh=_mesh,
    out_type=jax.ShapeDtypeStruct((_OROWS, 8 * NP), jnp.float32),
    scratch_types=[
        pltpu.VMEM((NCH, CHUNK_IDX), jnp.int32),
        pltpu.VMEM((NBUF, CHUNK_IDX, NP), jnp.float32),
        pltpu.VMEM((_ORPW, 8 * NP), jnp.float32),
        pltpu.VMEM((NP,), jnp.float32),
    ] + [pltpu.SemaphoreType.DMA] * NBUF,
    compiler_params=pltpu.CompilerParams(use_tc_tiling_on_sc=False),
)
def _bag_kernel(idx_hbm, p_hbm, b_hbm, out_hbm,
                idx_v, rows_v, out_v, b_v, *sems):
    wid = lax.axis_index("s") * NC + lax.axis_index("c")
    base = wid * NCH
    pltpu.sync_copy(idx_hbm.at[pl.ds(base, NCH)], idx_v)
    pltpu.sync_copy(b_hbm, b_v)
    bias = b_v[...]

    for bb in range(NBUF):
        pltpu.async_copy(p_hbm.at[idx_v.at[bb]], rows_v.at[bb], sems[bb])

    def body(g, _):
        for bb in range(NBUF):
            ch = g * NBUF + bb
            pltpu.make_async_copy(
                p_hbm.at[idx_v.at[bb]], rows_v.at[bb], sems[bb]
            ).wait()
            for bag in range(CB):
                r0 = bag * BAG_LEN
                acc0 = rows_v[bb, r0, :]
                acc1 = rows_v[bb, r0 + 1, :]
                for l in range(2, BAG_LEN, 2):
                    acc0 = acc0 + rows_v[bb, r0 + l, :]
                    acc1 = acc1 + rows_v[bb, r0 + l + 1, :]
                bg = bb * CB + bag
                out_v[(NBUF * CB // 8) * g + bg // 8, pl.ds((bg % 8) * NP, NP)] = (
                    acc0 + acc1 + bias
                )
            nxt = ch + NBUF

            @pl.when(nxt < NCH)
            def _():
                pltpu.async_copy(
                    p_hbm.at[idx_v.at[nxt]], rows_v.at[bb], sems[bb]
                )

        return 0

    lax.fori_loop(0, NCH // NBUF, body, 0)
    pltpu.sync_copy(out_v, out_hbm.at[pl.ds(wid * _ORPW, _ORPW)])


_BLK = 16384
_GRP = _BLK // 8
_GRID = (VOCAB + _BLK - 1) // _BLK


def _proj_block(tt_ref, w3_ref, o_ref):
    x = tt_ref[...]
    xr = jnp.concatenate(
        [x[:, g * _GRP:(g + 1) * _GRP] for g in range(8)], axis=0
    )
    o_ref[...] = jax.lax.dot_general(
        xr, w3_ref[...], (((0,), (0,)), ((), ())),
        preferred_element_type=jnp.float32,
        precision=jax.lax.Precision.DEFAULT,
    )


_VPAD = _GRID * _BLK


def _tc_project(table_t, w3):
    return pl.pallas_call(
        _proj_block,
        grid=(_GRID,),
        in_specs=[
            pl.BlockSpec((EMBED_DIM, _BLK), lambda i: (0, i)),
            pl.BlockSpec((8 * EMBED_DIM, 8 * NP), lambda i: (0, 0)),
        ],
        out_specs=pl.BlockSpec((_BLK // 8, 8 * NP), lambda i: (i, 0)),
        out_shape=jax.ShapeDtypeStruct((_VPAD // 8, 8 * NP), jnp.float32),
    )(table_t, w3)


def kernel(token_index, table, W, b):
    tok = token_index.astype(jnp.int32)
    tok = (tok // _BLK) * _BLK + (tok % _GRP) * 8 + (tok % _BLK) // _GRP
    idx = jnp.reshape(tok, (BATCH // CB, CHUNK_IDX))
    wt16 = (
        jnp.zeros((EMBED_DIM, NP), jnp.float32)
        .at[:, :NUM_CLASS]
        .set(W.T * jnp.float32(1.0 / BAG_LEN))
    )
    w3 = jnp.kron(jnp.eye(8, dtype=jnp.float32), wt16)
    b_pad = jnp.zeros((NP,), jnp.float32).at[:NUM_CLASS].set(b)
    proj = _tc_project(table.T, w3)
    p_rows = jnp.reshape(proj, (_VPAD, NP))
    out = _bag_kernel(idx, p_rows, b_pad)
    return jnp.reshape(out, (BATCH, NP))[:, :NUM_CLASS]


# speedup vs baseline: 1.5062x; 1.0077x over previous
import functools

import jax
import jax.numpy as jnp
from jax import lax
from jax.experimental import pallas as pl
from jax.experimental.pallas import tpu as pltpu
from jax.experimental.pallas import tpu_sc as plsc

VOCAB = 100000
EMBED_DIM = 64
NUM_CLASS = 2
BATCH = 4096
BAG_LEN = 50

NP = 16
NC = 2
NS = 16
LANES = 16
NW = NC * NS
BPW = BATCH // NW
CB = 2
CHUNK_IDX = CB * BAG_LEN
NCH = BPW // CB
NBUF = 8

_mesh = plsc.VectorSubcoreMesh(core_axis_name="c", subcore_axis_name="s")

_OROWS = BATCH // 8
_ORPW = _OROWS // NW


@functools.partial(
    pl.kernel,
    mesh=_mesh,
    out_type=jax.ShapeDtypeStruct((_OROWS, 8 * NP), jnp.float32),
    scratch_types=[
        pltpu.VMEM((NCH, CHUNK_IDX), jnp.int32),
        pltpu.VMEM((NBUF, CHUNK_IDX, NP), jnp.float32),
        pltpu.VMEM((_ORPW, 8 * NP), jnp.float32),
        pltpu.VMEM((NP,), jnp.float32),
    ] + [pltpu.SemaphoreType.DMA] * NBUF,
    compiler_params=pltpu.CompilerParams(use_tc_tiling_on_sc=False),
)
def _bag_kernel(idx_hbm, p_hbm, b_hbm, out_hbm,
                idx_v, rows_v, out_v, b_v, *sems):
    wid = lax.axis_index("s") * NC + lax.axis_index("c")
    base = wid * NCH
    pltpu.sync_copy(idx_hbm.at[pl.ds(base, NCH)], idx_v)
    pltpu.sync_copy(b_hbm, b_v)
    bias = b_v[...]

    for bb in range(NBUF):
        pltpu.async_copy(p_hbm.at[idx_v.at[bb]], rows_v.at[bb], sems[bb])

    def body(g, _):
        for bb in range(NBUF):
            ch = g * NBUF + bb
            pltpu.make_async_copy(
                p_hbm.at[idx_v.at[bb]], rows_v.at[bb], sems[bb]
            ).wait()
            for bag in range(CB):
                r0 = bag * BAG_LEN
                acc0 = rows_v[bb, r0, :]
                acc1 = rows_v[bb, r0 + 1, :]
                for l in range(2, BAG_LEN, 2):
                    acc0 = acc0 + rows_v[bb, r0 + l, :]
                    acc1 = acc1 + rows_v[bb, r0 + l + 1, :]
                bg = bb * CB + bag
                out_v[(NBUF * CB // 8) * g + bg // 8, pl.ds((bg % 8) * NP, NP)] = (
                    acc0 + acc1 + bias
                )
            nxt = ch + NBUF

            @pl.when(nxt < NCH)
            def _():
                pltpu.async_copy(
                    p_hbm.at[idx_v.at[nxt]], rows_v.at[bb], sems[bb]
                )

        return 0

    lax.fori_loop(0, NCH // NBUF, body, 0)
    pltpu.sync_copy(out_v, out_hbm.at[pl.ds(wid * _ORPW, _ORPW)])


_BLK = 32768
_GRP = _BLK // 8
_GRP_SHIFT = _GRP.bit_length() - 1
_GRID = (VOCAB + _BLK - 1) // _BLK


def _proj_block(tt_ref, w3_ref, o_ref):
    x = tt_ref[...]
    xr = jnp.concatenate(
        [x[:, g * _GRP:(g + 1) * _GRP] for g in range(8)], axis=0
    )
    o_ref[...] = jax.lax.dot_general(
        xr, w3_ref[...], (((0,), (0,)), ((), ())),
        preferred_element_type=jnp.float32,
        precision=jax.lax.Precision.DEFAULT,
    )


_VPAD = _GRID * _BLK


def _tc_project(table_t, w3):
    return pl.pallas_call(
        _proj_block,
        grid=(_GRID,),
        in_specs=[
            pl.BlockSpec((EMBED_DIM, _BLK), lambda i: (0, i)),
            pl.BlockSpec((8 * EMBED_DIM, 8 * NP), lambda i: (0, 0)),
        ],
        out_specs=pl.BlockSpec((_BLK // 8, 8 * NP), lambda i: (i, 0)),
        out_shape=jax.ShapeDtypeStruct((_VPAD // 8, 8 * NP), jnp.float32),
    )(table_t, w3)


def kernel(token_index, table, W, b):
    tok = token_index.astype(jnp.int32)
    tok = (
        (tok & ~(_BLK - 1))
        | ((tok & (_GRP - 1)) << 3)
        | ((tok >> _GRP_SHIFT) & 7)
    )
    idx = jnp.reshape(tok, (BATCH // CB, CHUNK_IDX))
    wt16 = (
        jnp.zeros((EMBED_DIM, NP), jnp.float32)
        .at[:, :NUM_CLASS]
        .set(W.T * jnp.float32(1.0 / BAG_LEN))
    )
    w3 = jnp.kron(jnp.eye(8, dtype=jnp.float32), wt16)
    b_pad = jnp.zeros((NP,), jnp.float32).at[:NUM_CLASS].set(b)
    proj = _tc_project(table.T, w3)
    p_rows = jnp.reshape(proj, (_VPAD, NP))
    out = _bag_kernel(idx, p_rows, b_pad)
    return jnp.reshape(out, (BATCH, NP))[:, :NUM_CLASS]


# speedup vs baseline: 1.5560x; 1.0330x over previous
import functools

import jax
import jax.numpy as jnp
from jax import lax
from jax.experimental import pallas as pl
from jax.experimental.pallas import tpu as pltpu
from jax.experimental.pallas import tpu_sc as plsc

VOCAB = 100000
EMBED_DIM = 64
NUM_CLASS = 2
BATCH = 4096
BAG_LEN = 50

NP = 16
NC = 2
NS = 16
LANES = 16
NW = NC * NS
BPW = BATCH // NW
CB = 2
CHUNK_IDX = CB * BAG_LEN
NCH = BPW // CB
NBUF = 8

_mesh = plsc.VectorSubcoreMesh(core_axis_name="c", subcore_axis_name="s")

_OROWS = BATCH // 8
_ORPW = _OROWS // NW


@functools.partial(
    pl.kernel,
    mesh=_mesh,
    out_type=jax.ShapeDtypeStruct((_OROWS, 8 * NP), jnp.float32),
    scratch_types=[
        pltpu.VMEM((NCH, CHUNK_IDX), jnp.int32),
        pltpu.VMEM((NBUF, CHUNK_IDX, NP), jnp.float32),
        pltpu.VMEM((_ORPW, 8 * NP), jnp.float32),
        pltpu.VMEM((NP,), jnp.float32),
    ] + [pltpu.SemaphoreType.DMA] * NBUF,
    compiler_params=pltpu.CompilerParams(use_tc_tiling_on_sc=False),
)
def _bag_kernel(idx_hbm, p_hbm, b_hbm, out_hbm,
                idx_v, rows_v, out_v, b_v, *sems):
    wid = lax.axis_index("s") * NC + lax.axis_index("c")
    base = wid * NCH
    pltpu.sync_copy(idx_hbm.at[pl.ds(base, NCH)], idx_v)
    pltpu.sync_copy(b_hbm, b_v)
    bias = b_v[...]

    for bb in range(NBUF):
        pltpu.async_copy(p_hbm.at[idx_v.at[bb]], rows_v.at[bb], sems[bb])

    def body(g, _):
        for bb in range(NBUF):
            ch = g * NBUF + bb
            pltpu.make_async_copy(
                p_hbm.at[idx_v.at[bb]], rows_v.at[bb], sems[bb]
            ).wait()
            for bag in range(CB):
                r0 = bag * BAG_LEN
                acc0 = rows_v[bb, r0, :]
                acc1 = rows_v[bb, r0 + 1, :]
                for l in range(2, BAG_LEN, 2):
                    acc0 = acc0 + rows_v[bb, r0 + l, :]
                    acc1 = acc1 + rows_v[bb, r0 + l + 1, :]
                bg = bb * CB + bag
                out_v[(NBUF * CB // 8) * g + bg // 8, pl.ds((bg % 8) * NP, NP)] = (
                    acc0 + acc1 + bias
                )
            nxt = ch + NBUF

            @pl.when(nxt < NCH)
            def _():
                pltpu.async_copy(
                    p_hbm.at[idx_v.at[nxt]], rows_v.at[bb], sems[bb]
                )

        return 0

    lax.fori_loop(0, NCH // NBUF, body, 0)
    pltpu.sync_copy(out_v, out_hbm.at[pl.ds(wid * _ORPW, _ORPW)])


_BLK = 65536
_GRP = _BLK // 8
_GRP_SHIFT = _GRP.bit_length() - 1
_GRID = (VOCAB + _BLK - 1) // _BLK


def _proj_block(tt_ref, w3_ref, o_ref):
    x = tt_ref[...]
    xr = jnp.concatenate(
        [x[:, g * _GRP:(g + 1) * _GRP] for g in range(8)], axis=0
    )
    o_ref[...] = jax.lax.dot_general(
        xr, w3_ref[...], (((0,), (0,)), ((), ())),
        preferred_element_type=jnp.float32,
        precision=jax.lax.Precision.DEFAULT,
    )


_VPAD = _GRID * _BLK


def _tc_project(table_t, w3):
    return pl.pallas_call(
        _proj_block,
        grid=(_GRID,),
        in_specs=[
            pl.BlockSpec((EMBED_DIM, _BLK), lambda i: (0, i)),
            pl.BlockSpec((8 * EMBED_DIM, 8 * NP), lambda i: (0, 0)),
        ],
        out_specs=pl.BlockSpec((_BLK // 8, 8 * NP), lambda i: (i, 0)),
        out_shape=jax.ShapeDtypeStruct((_VPAD // 8, 8 * NP), jnp.float32),
    )(table_t, w3)


def kernel(token_index, table, W, b):
    tok = token_index.astype(jnp.int32)
    tok = (
        (tok & ~(_BLK - 1))
        | ((tok & (_GRP - 1)) << 3)
        | ((tok >> _GRP_SHIFT) & 7)
    )
    idx = jnp.reshape(tok, (BATCH // CB, CHUNK_IDX))
    wt16 = jnp.pad(
        W.T * jnp.float32(1.0 / BAG_LEN), ((0, 0), (0, NP - NUM_CLASS))
    )
    w3 = jnp.kron(jnp.eye(8, dtype=jnp.float32), wt16)
    b_pad = jnp.pad(b, (0, NP - NUM_CLASS))
    proj = _tc_project(table.T, w3)
    p_rows = jnp.reshape(proj, (_VPAD, NP))
    out = _bag_kernel(idx, p_rows, b_pad)
    return jnp.reshape(out, (BATCH, NP))[:, :NUM_CLASS]
